# baseline (speedup 1.0000x reference)
import functools

import jax
import jax.numpy as jnp
from jax import lax
from jax.experimental import pallas as pl
from jax.experimental.pallas import tpu as pltpu
from jax.experimental.pallas import tpu_sc as plsc

N = 100000
E = 3200000
F_IN = 128
C = 16

NC = 2
NS = 16
NW = NC * NS

CH = 128
CPW = 782
TOTCH = NW * CPW
E_PAD = TOTCH * CH
NP = N + 16
RPT = NP // NS
ORPT = N // NS
GS = 5
IB = 2 * GS
NIB = CPW // IB
IBT = CPW - NIB * IB
ZR = 133

DEG_CB = 10000
DEG_EPW = E // NW
DEG_NB = DEG_EPW // DEG_CB

B = 2000
GRID = N // B


def _vmesh():
    return plsc.VectorSubcoreMesh(core_axis_name="c", subcore_axis_name="s")


def _sc_degree(col):
    @functools.partial(
        pl.kernel,
        out_type=jax.ShapeDtypeStruct((GRID, NW, B), jnp.float32),
        mesh=_vmesh(),
        compiler_params=pltpu.CompilerParams(needs_layout_passes=False, use_tc_tiling_on_sc=False),
        scratch_types=[
            pltpu.VMEM((N,), jnp.float32),
            pltpu.VMEM((2, DEG_CB), jnp.int32),
            pltpu.SemaphoreType.DMA,
            pltpu.SemaphoreType.DMA,
        ],
    )
    def k(col_hbm, out_hbm, deg_v, colbuf_v, dsa, dsb):
        wid = lax.axis_index("s") * NC + lax.axis_index("c")
        base = wid * DEG_EPW
        dsem = (dsa, dsb)

        def zero_body(i, _):
            deg_v[pl.ds(i * 16, 16)] = jnp.zeros((16,), jnp.float32)
            return 0

        lax.fori_loop(0, N // 16, zero_body, 0)

        ones = jnp.ones((16,), jnp.float32)

        pltpu.async_copy(col_hbm.at[pl.ds(base, DEG_CB)], colbuf_v.at[0], dsa)

        def blk2_body(p, _):
            for iset in range(2):
                b = 2 * p + iset

                @pl.when(b + 1 < DEG_NB)
                def _():
                    pltpu.async_copy(
                        col_hbm.at[pl.ds(base + (b + 1) * DEG_CB, DEG_CB)],
                        colbuf_v.at[1 - iset], dsem[1 - iset])

                pltpu.make_async_copy(col_hbm.at[pl.ds(0, DEG_CB)],
                                      colbuf_v.at[iset], dsem[iset]).wait()

                def inner(i, _):
                    idx = colbuf_v[iset, pl.ds(i * 16, 16)]
                    plsc.addupdate_scatter(deg_v, [idx], ones)
                    return 0

                lax.fori_loop(0, DEG_CB // 16, inner, 0)
            return 0

        lax.fori_loop(0, DEG_NB // 2, blk2_body, 0)

        def out_body(i, _):
            pltpu.sync_copy(deg_v.at[pl.ds(i * B, B)], out_hbm.at[i, wid, :])
            return 0

        lax.fori_loop(0, GRID, out_body, 0)

    return k(col)


def _sc_edge_pass(xp, rowc, colc):
    @functools.partial(
        pl.kernel,
        out_type=jax.ShapeDtypeStruct((NC, N, 16), jnp.float32),
        mesh=_vmesh(),
        compiler_params=pltpu.CompilerParams(needs_layout_passes=False, use_tc_tiling_on_sc=False),
        scratch_types=[
            pltpu.VMEM((2, IB, CH), jnp.int32),
            pltpu.VMEM((2, IB, CH), jnp.int32),
            pltpu.VMEM((2, GS * CH, 16), jnp.float32),
            pltpu.VMEM((1, CH), jnp.int32),
            pltpu.VMEM((ZR, 16), jnp.float32),
            pltpu.VMEM_SHARED((NP, 16), jnp.float32),
            pltpu.SemaphoreType.DMA,
            pltpu.SemaphoreType.DMA,
            pltpu.SemaphoreType.DMA,
            pltpu.SemaphoreType.DMA,
            pltpu.SemaphoreType.DMA,
            pltpu.SemaphoreType.DMA,
        ],
    )
    def k(xp_hbm, row_hbm, col_hbm, out_hbm, rowb, colb, bufs, dumidx, zb,
          acc, gsa, gsb, ssa, ssb, isa, isb):
        cid = lax.axis_index("c")
        sid = lax.axis_index("s")
        wid = sid * NC + cid
        ssem = (ssa, ssb)
        gsem = (gsa, gsb)

        def zb_body(i, _):
            zb[i, :] = jnp.zeros((16,), jnp.float32)
            return 0

        lax.fori_loop(0, ZR, zb_body, 0)

        def zcopy(t, _):
            pltpu.sync_copy(zb, acc.at[pl.ds(sid * RPT + t * ZR, ZR), :])
            return 0

        lax.fori_loop(0, RPT // ZR, zcopy, 0)

        for i in range(CH // 16):
            dumidx[0, pl.ds(i * 16, 16)] = jnp.full((16,), N, jnp.int32)
        def buf_zero(i, _):
            h = i // (GS * CH)
            r = i % (GS * CH)
            bufs[h, r, :] = jnp.zeros((16,), jnp.float32)
            return 0

        lax.fori_loop(0, 2 * GS * CH, buf_zero, 0)
        plsc.subcore_barrier()

        def prime(i, _):
            pltpu.async_copy(bufs.at[0, pl.ds(0, CH), :],
                             acc.at[dumidx.at[0]], ssa, add=True)
            pltpu.async_copy(bufs.at[1, pl.ds(0, CH), :],
                             acc.at[dumidx.at[0]], ssb, add=True)
            return 0

        lax.fori_loop(0, GS, prime, 0)

        base_ch = wid * CPW
        isem = (isa, isb)

        def stage(b, s, sem):
            c0 = base_ch + b * IB
            pltpu.async_copy(row_hbm.at[pl.ds(c0, IB), :], rowb.at[s], sem)
            pltpu.async_copy(col_hbm.at[pl.ds(c0, IB), :], colb.at[s], sem)

        stage(0, 0, isa)

        def blk2(p, _):
            for iset in range(2):
                b = 2 * p + iset

                @pl.when(b + 1 < NIB)
                def _():
                    stage(b + 1, 1 - iset, isem[1 - iset])

                pltpu.make_async_copy(row_hbm.at[pl.ds(0, IB), :],
                                      rowb.at[iset], isem[iset]).wait()
                pltpu.make_async_copy(col_hbm.at[pl.ds(0, IB), :],
                                      colb.at[iset], isem[iset]).wait()
                for half in range(2):
                    pltpu.make_async_copy(xp_hbm.at[pl.ds(0, GS * CH), :],
                                          bufs.at[half], ssem[half]).wait()
                    gd = []
                    for g in range(GS):
                        gd.append(pltpu.async_copy(
                            xp_hbm.at[rowb.at[iset, half * GS + g]],
                            bufs.at[half, pl.ds(g * CH, CH), :], gsem[half]))
                    for g in range(GS):
                        gd[g].wait()
                        pltpu.async_copy(bufs.at[half, pl.ds(g * CH, CH), :],
                                         acc.at[colb.at[iset, half * GS + g]],
                                         ssem[half], add=True)
            return 0

        lax.fori_loop(0, NIB // 2, blk2, 0)

        for half in range(2):
            pltpu.make_async_copy(xp_hbm.at[pl.ds(0, GS * CH), :],
                                  bufs.at[half], ssem[half]).wait()

        c0t = base_ch + NIB * IB
        pltpu.sync_copy(row_hbm.at[pl.ds(c0t, IBT), :],
                        rowb.at[0, pl.ds(0, IBT), :])
        pltpu.sync_copy(col_hbm.at[pl.ds(c0t, IBT), :],
                        colb.at[0, pl.ds(0, IBT), :])
        def tail(t, _):
            g = pltpu.async_copy(xp_hbm.at[rowb.at[0, t]],
                                 bufs.at[0, pl.ds(0, CH), :], gsa)
            g.wait()
            s = pltpu.async_copy(bufs.at[0, pl.ds(0, CH), :],
                                 acc.at[colb.at[0, t]], ssa, add=True)
            s.wait()
            return 0

        lax.fori_loop(0, IBT, tail, 0)
        plsc.subcore_barrier()
        pltpu.sync_copy(acc.at[pl.ds(sid * ORPT, ORPT), :],
                        out_hbm.at[cid, pl.ds(sid * ORPT, ORPT), :])

    return k(xp, rowc, colc)


NF = N // 8


def _bcast_rows(v_row):
    ones16 = jnp.ones((1, 16), jnp.float32)
    return lax.dot_general(v_row, ones16, (((0,), (0,)), ((), ())),
                           preferred_element_type=jnp.float32)


def _k1a_body(x_ref, wnn_ref, bnn_ref, gi_ref, wgn_ref, bgn_ref, xw_ref):
    gvec = jnp.dot(gi_ref[...], wgn_ref[...],
                   preferred_element_type=jnp.float32) + bgn_ref[...]
    xw_ref[...] = jnp.dot(x_ref[...], wnn_ref[...],
                          preferred_element_type=jnp.float32) + bnn_ref[...] + gvec


def _tc_k1a(x, wnn, bnn, gi, wgn, bgn):
    return pl.pallas_call(
        _k1a_body,
        grid=(GRID,),
        in_specs=[
            pl.BlockSpec((B, F_IN), lambda i: (i, 0)),
            pl.BlockSpec((F_IN, 16), lambda i: (0, 0)),
            pl.BlockSpec((1, 16), lambda i: (0, 0)),
            pl.BlockSpec((1, 16), lambda i: (0, 0)),
            pl.BlockSpec((16, 16), lambda i: (0, 0)),
            pl.BlockSpec((1, 16), lambda i: (0, 0)),
        ],
        out_specs=pl.BlockSpec((B, 16), lambda i: (i, 0)),
        out_shape=jax.ShapeDtypeStruct((N, 16), jnp.float32),
    )(x, wnn, bnn, gi, wgn, bgn)


def _k1b_body(xw_ref, degp_ref, xp_ref, dinvb_ref):
    deg = jnp.sum(degp_ref[0], axis=0, keepdims=True) + 1.0
    dinvb = _bcast_rows(lax.rsqrt(deg))
    dinvb_ref[...] = dinvb
    xp_ref[...] = xw_ref[...] * dinvb


def _tc_k1b(xw, deg_parts):
    return pl.pallas_call(
        _k1b_body,
        grid=(GRID,),
        in_specs=[
            pl.BlockSpec((B, 16), lambda i: (i, 0)),
            pl.BlockSpec((1, NW, B), lambda i: (i, 0, 0)),
        ],
        out_specs=[
            pl.BlockSpec((B, 16), lambda i: (i, 0)),
            pl.BlockSpec((B, 16), lambda i: (i, 0)),
        ],
        out_shape=[
            jax.ShapeDtypeStruct((N, 16), jnp.float32),
            jax.ShapeDtypeStruct((N, 16), jnp.float32),
        ],
    )(xw, deg_parts)


def _k4a_body(s_ref, xp_ref, dinv_ref, h_ref, gmax_ref):
    out1 = dinv_ref[...] * (s_ref[0:NF] + s_ref[NF:2 * NF] + xp_ref[...])
    h_ref[...] = jnp.maximum(out1, 0.0)
    gmax_ref[...] = jnp.max(out1, axis=0, keepdims=True)


def _tc_k4a(sf, xpf, dinvf):
    return pl.pallas_call(
        _k4a_body,
        out_shape=[
            jax.ShapeDtypeStruct((NF, 128), jnp.float32),
            jax.ShapeDtypeStruct((1, 128), jnp.float32),
        ],
    )(sf, xpf, dinvf)


def _k4b_body(h_ref, dinv_ref, gmax_ref, gi_ref, wgg_ref, bgg_ref, wng_ref,
              bng_ref, wgn_ref, bgn_ref, wk_ref, bt_ref, xp2_ref):
    g128 = gmax_ref[...]
    gmax = g128[:, 0:16]
    for j in range(1, 8):
        gmax = jnp.maximum(gmax, g128[:, 16 * j:16 * j + 16])
    glob1 = (jnp.dot(gi_ref[...], wgg_ref[...],
                     preferred_element_type=jnp.float32) + bgg_ref[...]
             + jnp.dot(gmax, wng_ref[...],
                       preferred_element_type=jnp.float32) + bng_ref[...])
    gvec2 = jnp.dot(glob1, wgn_ref[...],
                    preferred_element_type=jnp.float32) + bgn_ref[...]
    gvec2t = jnp.concatenate([gvec2] * 8, axis=1)
    xw2 = jnp.dot(h_ref[...], wk_ref[...],
                  preferred_element_type=jnp.float32) + bt_ref[...] + gvec2t
    xp2_ref[...] = xw2 * dinv_ref[...]


def _tc_k4b(h, dinvf, gmax, gi, wgg, bgg, wng, bng, wgn, bgn, wk2, bt2):
    return pl.pallas_call(
        _k4b_body,
        out_shape=jax.ShapeDtypeStruct((NF, 128), jnp.float32),
    )(h, dinvf, gmax, gi, wgg, bgg, wng, bng, wgn, bgn, wk2, bt2)


def _k6_body(s_ref, xp_ref, dinv_ref, out_ref):
    z = dinv_ref[...] * (s_ref[0:NF] + s_ref[NF:2 * NF] + xp_ref[...])
    out_ref[...] = 1.0 / (1.0 + jnp.exp(-z))


def _tc_k6(sf, xpf, dinvf):
    return pl.pallas_call(
        _k6_body,
        out_shape=jax.ShapeDtypeStruct((NF, 128), jnp.float32),
    )(sf, xpf, dinvf)


def kernel(x, edge_index, W_nn1, b_nn1, W_gn1, b_gn1, W_gg1, b_gg1, W_ng1,
           b_ng1, W_nn2, b_nn2, W_gn2, b_gn2, W_gg2, b_gg2, W_ng2, b_ng2,
           glob_init):
    row = edge_index[0]
    col = edge_index[1]
    pad = E_PAD - E
    rowc = jnp.concatenate([row, jnp.zeros((pad,), jnp.int32)]).reshape(TOTCH, CH)
    colc = jnp.concatenate([col, jnp.full((pad,), N, jnp.int32)]).reshape(TOTCH, CH)

    wk2 = jnp.kron(jnp.eye(8, dtype=jnp.float32), W_nn2)
    bt2 = jnp.tile(b_nn2, 8).reshape(1, 128)

    deg_parts = _sc_degree(col)
    xw1 = _tc_k1a(x, W_nn1, b_nn1.reshape(1, 16), glob_init, W_gn1,
                  b_gn1.reshape(1, 16))
    xp1, dinvb = _tc_k1b(xw1, deg_parts)
    dinvf = dinvb.reshape(NF, 128)
    s1 = _sc_edge_pass(xp1, rowc, colc)
    h, gmax = _tc_k4a(s1.reshape(2 * NF, 128), xp1.reshape(NF, 128), dinvf)
    xp2f = _tc_k4b(h, dinvf, gmax, glob_init, W_gg1, b_gg1.reshape(1, 16),
                   W_ng1, b_ng1.reshape(1, 16), W_gn2, b_gn2.reshape(1, 16),
                   wk2, bt2)
    s2 = _sc_edge_pass(xp2f.reshape(N, 16), rowc, colc)
    return _tc_k6(s2.reshape(2 * NF, 128), xp2f, dinvf).reshape(N, 16)


# speedup vs baseline: 1.0352x; 1.0352x over previous
import functools

import jax
import jax.numpy as jnp
from jax import lax
from jax.experimental import pallas as pl
from jax.experimental.pallas import tpu as pltpu
from jax.experimental.pallas import tpu_sc as plsc

N = 100000
E = 3200000
F_IN = 128
C = 16

NC = 2
NS = 16
NW = NC * NS

CH = 128
CPW = 782
TOTCH = NW * CPW
E_PAD = TOTCH * CH
NP = N + 16
RPT = NP // NS
ORPT = N // NS
GS = 5
IB = 2 * GS
NIB = CPW // IB
IBT = CPW - NIB * IB
ZR = 133

DEG_CB = 10000
DEG_EPW = E // NW
DEG_NB = DEG_EPW // DEG_CB

B = 2000
GRID = N // B


def _vmesh():
    return plsc.VectorSubcoreMesh(core_axis_name="c", subcore_axis_name="s")


def _sc_degree(col):
    @functools.partial(
        pl.kernel,
        out_type=jax.ShapeDtypeStruct((GRID, NW, B), jnp.float32),
        mesh=_vmesh(),
        compiler_params=pltpu.CompilerParams(needs_layout_passes=False, use_tc_tiling_on_sc=False),
        scratch_types=[
            pltpu.VMEM((N,), jnp.float32),
            pltpu.VMEM((2, DEG_CB), jnp.int32),
            pltpu.SemaphoreType.DMA,
            pltpu.SemaphoreType.DMA,
        ],
    )
    def k(col_hbm, out_hbm, deg_v, colbuf_v, dsa, dsb):
        wid = lax.axis_index("s") * NC + lax.axis_index("c")
        base = wid * DEG_EPW
        dsem = (dsa, dsb)

        def zero_body(i, _):
            deg_v[pl.ds(i * 16, 16)] = jnp.zeros((16,), jnp.float32)
            return 0

        lax.fori_loop(0, N // 16, zero_body, 0)

        ones = jnp.ones((16,), jnp.float32)

        pltpu.async_copy(col_hbm.at[pl.ds(base, DEG_CB)], colbuf_v.at[0], dsa)

        def blk2_body(p, _):
            for iset in range(2):
                b = 2 * p + iset

                @pl.when(b + 1 < DEG_NB)
                def _():
                    pltpu.async_copy(
                        col_hbm.at[pl.ds(base + (b + 1) * DEG_CB, DEG_CB)],
                        colbuf_v.at[1 - iset], dsem[1 - iset])

                pltpu.make_async_copy(col_hbm.at[pl.ds(0, DEG_CB)],
                                      colbuf_v.at[iset], dsem[iset]).wait()

                def inner(i, _):
                    idx = colbuf_v[iset, pl.ds(i * 16, 16)]
                    plsc.addupdate_scatter(deg_v, [idx], ones)
                    return 0

                lax.fori_loop(0, DEG_CB // 16, inner, 0)
            return 0

        lax.fori_loop(0, DEG_NB // 2, blk2_body, 0)

        def out_body(i, _):
            pltpu.sync_copy(deg_v.at[pl.ds(i * B, B)], out_hbm.at[i, wid, :])
            return 0

        lax.fori_loop(0, GRID, out_body, 0)

    return k(col)


def _sc_edge_pass(xp, rowc, colc):
    @functools.partial(
        pl.kernel,
        out_type=jax.ShapeDtypeStruct((NC, N, 16), jnp.float32),
        mesh=_vmesh(),
        compiler_params=pltpu.CompilerParams(needs_layout_passes=False, use_tc_tiling_on_sc=False),
        scratch_types=[
            pltpu.VMEM((2, IB, CH), jnp.int32),
            pltpu.VMEM((2, IB, CH), jnp.int32),
            pltpu.VMEM((2, GS * CH, 16), jnp.float32),
            pltpu.VMEM((1, CH), jnp.int32),
            pltpu.VMEM((ZR, 16), jnp.float32),
            pltpu.VMEM_SHARED((NP, 16), jnp.float32),
            pltpu.SemaphoreType.DMA,
            pltpu.SemaphoreType.DMA,
            pltpu.SemaphoreType.DMA,
            pltpu.SemaphoreType.DMA,
            pltpu.SemaphoreType.DMA,
            pltpu.SemaphoreType.DMA,
        ],
    )
    def k(xp_hbm, row_hbm, col_hbm, out_hbm, rowb, colb, bufs, dumidx, zb,
          acc, gsa, gsb, ssa, ssb, isa, isb):
        cid = lax.axis_index("c")
        sid = lax.axis_index("s")
        wid = sid * NC + cid
        ssem = (ssa, ssb)
        gsem = (gsa, gsb)

        def zb_body(i, _):
            zb[i, :] = jnp.zeros((16,), jnp.float32)
            return 0

        lax.fori_loop(0, ZR, zb_body, 0)

        def zcopy(t, _):
            pltpu.sync_copy(zb, acc.at[pl.ds(sid * RPT + t * ZR, ZR), :])
            return 0

        lax.fori_loop(0, RPT // ZR, zcopy, 0)

        for i in range(CH // 16):
            dumidx[0, pl.ds(i * 16, 16)] = jnp.full((16,), N, jnp.int32)
        def buf_zero(i, _):
            h = i // (GS * CH)
            r = i % (GS * CH)
            bufs[h, r, :] = jnp.zeros((16,), jnp.float32)
            return 0

        lax.fori_loop(0, 2 * GS * CH, buf_zero, 0)
        plsc.subcore_barrier()

        def prime(i, _):
            pltpu.async_copy(bufs.at[0, pl.ds(0, CH), :],
                             acc.at[dumidx.at[0]], ssa, add=True)
            pltpu.async_copy(bufs.at[1, pl.ds(0, CH), :],
                             acc.at[dumidx.at[0]], ssb, add=True)
            return 0

        lax.fori_loop(0, GS, prime, 0)

        base_ch = wid * CPW
        isem = (isa, isb)

        def stage(b, s, sem):
            c0 = base_ch + b * IB
            pltpu.async_copy(row_hbm.at[pl.ds(c0, IB), :], rowb.at[s], sem)
            pltpu.async_copy(col_hbm.at[pl.ds(c0, IB), :], colb.at[s], sem)

        stage(0, 0, isa)

        def blk2(p, _):
            for iset in range(2):
                b = 2 * p + iset

                @pl.when(b + 1 < NIB)
                def _():
                    stage(b + 1, 1 - iset, isem[1 - iset])

                pltpu.make_async_copy(row_hbm.at[pl.ds(0, IB), :],
                                      rowb.at[iset], isem[iset]).wait()
                pltpu.make_async_copy(col_hbm.at[pl.ds(0, IB), :],
                                      colb.at[iset], isem[iset]).wait()
                for half in range(2):
                    pltpu.make_async_copy(xp_hbm.at[pl.ds(0, GS * CH), :],
                                          bufs.at[half], ssem[half]).wait()
                    gd = []
                    for g in range(GS):
                        gd.append(pltpu.async_copy(
                            xp_hbm.at[rowb.at[iset, half * GS + g]],
                            bufs.at[half, pl.ds(g * CH, CH), :], gsem[half]))
                    for g in range(GS):
                        gd[g].wait()
                        pltpu.async_copy(bufs.at[half, pl.ds(g * CH, CH), :],
                                         acc.at[colb.at[iset, half * GS + g]],
                                         ssem[half], add=True)
            return 0

        lax.fori_loop(0, NIB // 2, blk2, 0)

        for half in range(2):
            pltpu.make_async_copy(xp_hbm.at[pl.ds(0, GS * CH), :],
                                  bufs.at[half], ssem[half]).wait()

        c0t = base_ch + NIB * IB
        pltpu.sync_copy(row_hbm.at[pl.ds(c0t, IBT), :],
                        rowb.at[0, pl.ds(0, IBT), :])
        pltpu.sync_copy(col_hbm.at[pl.ds(c0t, IBT), :],
                        colb.at[0, pl.ds(0, IBT), :])
        def tail(t, _):
            g = pltpu.async_copy(xp_hbm.at[rowb.at[0, t]],
                                 bufs.at[0, pl.ds(0, CH), :], gsa)
            g.wait()
            s = pltpu.async_copy(bufs.at[0, pl.ds(0, CH), :],
                                 acc.at[colb.at[0, t]], ssa, add=True)
            s.wait()
            return 0

        lax.fori_loop(0, IBT, tail, 0)
        plsc.subcore_barrier()
        pltpu.sync_copy(acc.at[pl.ds(sid * ORPT, ORPT), :],
                        out_hbm.at[cid, pl.ds(sid * ORPT, ORPT), :])

    return k(xp, rowc, colc)


NF = N // 8


def _bcast_rows(v_row):
    ones16 = jnp.ones((1, 16), jnp.float32)
    return lax.dot_general(v_row, ones16, (((0,), (0,)), ((), ())),
                           preferred_element_type=jnp.float32)


def _k1a_body(x_ref, wk_ref, bnn_ref, gi_ref, wgn_ref, bgn_ref, xw_ref):
    k = pl.program_id(0)
    part = jnp.dot(x_ref[...], wk_ref[...],
                   preferred_element_type=jnp.float32)

    @pl.when(k == 0)
    def _():
        gvec = jnp.dot(gi_ref[...], wgn_ref[...],
                       preferred_element_type=jnp.float32) + bgn_ref[...]
        gt = jnp.concatenate([gvec + bnn_ref[...]] * 8, axis=1)
        xw_ref[...] = part + gt

    @pl.when(k > 0)
    def _():
        xw_ref[...] = xw_ref[...] + part


def _tc_k1a(x2, wk1, bnn, gi, wgn, bgn):
    return pl.pallas_call(
        _k1a_body,
        grid=(8,),
        in_specs=[
            pl.BlockSpec((NF, F_IN), lambda k: (0, k)),
            pl.BlockSpec((F_IN, 128), lambda k: (k, 0)),
            pl.BlockSpec((1, 16), lambda k: (0, 0)),
            pl.BlockSpec((1, 16), lambda k: (0, 0)),
            pl.BlockSpec((16, 16), lambda k: (0, 0)),
            pl.BlockSpec((1, 16), lambda k: (0, 0)),
        ],
        out_specs=pl.BlockSpec((NF, 128), lambda k: (0, 0)),
        out_shape=jax.ShapeDtypeStruct((NF, 128), jnp.float32),
    )(x2, wk1, bnn, gi, wgn, bgn)


def _k1bx_body(degp_ref, dinv_ref):
    deg = jnp.sum(degp_ref[0], axis=0, keepdims=True) + 1.0
    dinv_ref[0] = lax.rsqrt(deg)


def _tc_k1bx(deg_parts):
    return pl.pallas_call(
        _k1bx_body,
        grid=(GRID,),
        in_specs=[pl.BlockSpec((1, NW, B), lambda i: (i, 0, 0))],
        out_specs=pl.BlockSpec((1, 1, B), lambda i: (i, 0, 0)),
        out_shape=jax.ShapeDtypeStruct((GRID, 1, B), jnp.float32),
    )(deg_parts)


def _k1by_body(xw_ref, dinv8_ref, r8_ref, xp_ref, dinv_ref):
    dinvf = jnp.dot(dinv8_ref[...], r8_ref[...],
                    preferred_element_type=jnp.float32)
    dinv_ref[...] = dinvf
    xp_ref[...] = xw_ref[...] * dinvf


def _tc_k1by(xw, dinv8, r8):
    return pl.pallas_call(
        _k1by_body,
        out_shape=[
            jax.ShapeDtypeStruct((NF, 128), jnp.float32),
            jax.ShapeDtypeStruct((NF, 128), jnp.float32),
        ],
    )(xw, dinv8, r8)


def _k4a_body(s_ref, xp_ref, dinv_ref, h_ref, gmax_ref):
    out1 = dinv_ref[...] * (s_ref[0:NF] + s_ref[NF:2 * NF] + xp_ref[...])
    h_ref[...] = jnp.maximum(out1, 0.0)
    gmax_ref[...] = jnp.max(out1, axis=0, keepdims=True)


def _tc_k4a(sf, xpf, dinvf):
    return pl.pallas_call(
        _k4a_body,
        out_shape=[
            jax.ShapeDtypeStruct((NF, 128), jnp.float32),
            jax.ShapeDtypeStruct((1, 128), jnp.float32),
        ],
    )(sf, xpf, dinvf)


def _k4b_body(h_ref, dinv_ref, gmax_ref, gi_ref, wgg_ref, bgg_ref, wng_ref,
              bng_ref, wgn_ref, bgn_ref, wk_ref, bt_ref, xp2_ref):
    g128 = gmax_ref[...]
    gmax = g128[:, 0:16]
    for j in range(1, 8):
        gmax = jnp.maximum(gmax, g128[:, 16 * j:16 * j + 16])
    glob1 = (jnp.dot(gi_ref[...], wgg_ref[...],
                     preferred_element_type=jnp.float32) + bgg_ref[...]
             + jnp.dot(gmax, wng_ref[...],
                       preferred_element_type=jnp.float32) + bng_ref[...])
    gvec2 = jnp.dot(glob1, wgn_ref[...],
                    preferred_element_type=jnp.float32) + bgn_ref[...]
    gvec2t = jnp.concatenate([gvec2] * 8, axis=1)
    xw2 = jnp.dot(h_ref[...], wk_ref[...],
                  preferred_element_type=jnp.float32) + bt_ref[...] + gvec2t
    xp2_ref[...] = xw2 * dinv_ref[...]


def _tc_k4b(h, dinvf, gmax, gi, wgg, bgg, wng, bng, wgn, bgn, wk2, bt2):
    return pl.pallas_call(
        _k4b_body,
        out_shape=jax.ShapeDtypeStruct((NF, 128), jnp.float32),
    )(h, dinvf, gmax, gi, wgg, bgg, wng, bng, wgn, bgn, wk2, bt2)


def _k6_body(s_ref, xp_ref, dinv_ref, out_ref):
    z = dinv_ref[...] * (s_ref[0:NF] + s_ref[NF:2 * NF] + xp_ref[...])
    out_ref[...] = 1.0 / (1.0 + jnp.exp(-z))


def _tc_k6(sf, xpf, dinvf):
    return pl.pallas_call(
        _k6_body,
        out_shape=jax.ShapeDtypeStruct((NF, 128), jnp.float32),
    )(sf, xpf, dinvf)


def kernel(x, edge_index, W_nn1, b_nn1, W_gn1, b_gn1, W_gg1, b_gg1, W_ng1,
           b_ng1, W_nn2, b_nn2, W_gn2, b_gn2, W_gg2, b_gg2, W_ng2, b_ng2,
           glob_init):
    row = edge_index[0]
    col = edge_index[1]
    pad = E_PAD - E
    rowc = jnp.concatenate([row, jnp.zeros((pad,), jnp.int32)]).reshape(TOTCH, CH)
    colc = jnp.concatenate([col, jnp.full((pad,), N, jnp.int32)]).reshape(TOTCH, CH)

    wk2 = jnp.kron(jnp.eye(8, dtype=jnp.float32), W_nn2)
    bt2 = jnp.tile(b_nn2, 8).reshape(1, 128)

    wk1 = jnp.kron(jnp.eye(8, dtype=jnp.float32), W_nn1)
    r8 = jnp.kron(jnp.eye(8, dtype=jnp.float32),
                  jnp.ones((1, 16), jnp.float32))
    x2 = x.reshape(NF, 8 * F_IN)

    deg_parts = _sc_degree(col)
    xw1 = _tc_k1a(x2, wk1, b_nn1.reshape(1, 16), glob_init, W_gn1,
                  b_gn1.reshape(1, 16))
    dinv8 = _tc_k1bx(deg_parts).reshape(NF, 8)
    xp1f, dinvf = _tc_k1by(xw1, dinv8, r8)
    s1 = _sc_edge_pass(xp1f.reshape(N, 16), rowc, colc)
    h, gmax = _tc_k4a(s1.reshape(2 * NF, 128), xp1f, dinvf)
    xp2f = _tc_k4b(h, dinvf, gmax, glob_init, W_gg1, b_gg1.reshape(1, 16),
                   W_ng1, b_ng1.reshape(1, 16), W_gn2, b_gn2.reshape(1, 16),
                   wk2, bt2)
    s2 = _sc_edge_pass(xp2f.reshape(N, 16), rowc, colc)
    return _tc_k6(s2.reshape(2 * NF, 128), xp2f, dinvf).reshape(N, 16)
